# baseline (speedup 1.0000x reference)
import jax
import jax.numpy as jnp
from jax import lax
from jax.experimental import pallas as pl
from jax.experimental.pallas import tpu as pltpu
from jax.experimental.pallas import tpu_sc as plsc

_EMB_DIM = 32
_BATCH = 16384
_NC = 2
_NS = 16
_NW = _NC * _NS
_B_PER_W = _BATCH // _NW
_CHUNK = 128
_NCHUNK = _B_PER_W // _CHUNK
_GROUPS = _B_PER_W // 16
_NBUF = 8


def _mf_body(uidx_hbm, iidx_hbm, ut_hbm, it_hbm, ubias_hbm, ibias_hbm,
             out_hbm,
             uidx_v, iidx_v, ublk, iblk, bu_v, bi_v, pbuf, out_v,
             usem, isem, bsem):
    wid = lax.axis_index("s") * _NC + lax.axis_index("c")

    pltpu.sync_copy(uidx_hbm.at[wid], uidx_v)
    pltpu.sync_copy(iidx_hbm.at[wid], iidx_v)

    bias_copies = []
    for j in range(_NCHUNK):
        sl = pl.ds(j * _CHUNK, _CHUNK)
        bias_copies.append(
            pltpu.async_copy(ubias_hbm.at[uidx_v.at[sl]], bu_v.at[sl], bsem))
        bias_copies.append(
            pltpu.async_copy(ibias_hbm.at[iidx_v.at[sl]], bi_v.at[sl], bsem))

    iota16 = lax.iota(jnp.int32, 16)

    def _issue(ucol, icol, slot):
        uc = pl.multiple_of(ucol, 128)
        ic = pl.multiple_of(icol, 128)
        for r in range(4):
            rows = pl.ds(8 * r, 8)
            pltpu.async_copy(ut_hbm.at[rows, pl.ds(uc, 128)],
                             ublk.at[slot, rows], usem.at[slot])
            pltpu.async_copy(it_hbm.at[rows, pl.ds(ic, 128)],
                             iblk.at[slot, rows], isem.at[slot])

    def _drain(slot):
        for r in range(4):
            rows = pl.ds(8 * r, 8)
            pltpu.make_async_copy(ut_hbm.at[rows, pl.ds(0, 128)],
                                  ublk.at[slot, rows], usem.at[slot]).wait()
            pltpu.make_async_copy(it_hbm.at[rows, pl.ds(0, 128)],
                                  iblk.at[slot, rows], isem.at[slot]).wait()

    def _cols_lanes(g):
        uvec = uidx_v[pl.ds(g * 16, 16)]
        ivec = iidx_v[pl.ds(g * 16, 16)]
        return ((uvec >> 7) * 128, uvec & 127, (ivec >> 7) * 128, ivec & 127)

    ucol0, _, icol0, _ = _cols_lanes(0)
    for b in range(_NBUF):
        _issue(ucol0[b], icol0[b], b % _NBUF)

    def group_body(g, carry):
        ucolv, ulanev, icolv, ilanev = _cols_lanes(g)
        for b in range(16):
            slot = b % _NBUF
            _drain(slot)
            ul = jnp.broadcast_to(ulanev[b], (16,))
            il = jnp.broadcast_to(ilanev[b], (16,))
            u0 = plsc.load_gather(ublk.at[slot], [iota16, ul])
            u1 = plsc.load_gather(ublk.at[slot], [iota16 + 16, ul])
            i0 = plsc.load_gather(iblk.at[slot], [iota16, il])
            i1 = plsc.load_gather(iblk.at[slot], [iota16 + 16, il])
            pbuf[pl.ds(b * 16, 16)] = u0 * i0 + u1 * i1

            if b < 16 - _NBUF:
                _issue(ucolv[b + _NBUF], icolv[b + _NBUF], slot)
            else:

                @pl.when(g < _GROUPS - 1)
                def _():
                    ucolvn, _, icolvn, _ = _cols_lanes(g + 1)
                    _issue(ucolvn[b + _NBUF - 16], icolvn[b + _NBUF - 16],
                           slot)

        acc = bu_v[pl.ds(g * 16, 16)] + bi_v[pl.ds(g * 16, 16)]
        for c in range(16):
            acc = acc + plsc.load_gather(pbuf, [iota16 * 16 + c])
        out_v[pl.ds(g * 16, 16)] = acc
        return carry

    for c in bias_copies:
        c.wait()

    lax.fori_loop(0, _GROUPS, group_body, 0)

    pltpu.sync_copy(out_v, out_hbm.at[pl.ds(wid * _B_PER_W, _B_PER_W)])


def kernel(edge_index, user_emb, item_emb, user_bias, item_bias):
    ut = user_emb.T
    it = item_emb.T
    uidx = edge_index[0].astype(jnp.int32).reshape(_NW, _B_PER_W)
    iidx = edge_index[1].astype(jnp.int32).reshape(_NW, _B_PER_W)
    ub = user_bias.reshape(-1)
    ib = item_bias.reshape(-1)

    mesh = plsc.VectorSubcoreMesh(core_axis_name="c", subcore_axis_name="s")
    run = pl.kernel(
        _mf_body,
        mesh=mesh,
        compiler_params=pltpu.CompilerParams(
            needs_layout_passes=False,
            use_tc_tiling_on_sc=True,
        ),
        out_type=jax.ShapeDtypeStruct((_BATCH,), jnp.float32),
        scratch_types=[
            pltpu.VMEM((_B_PER_W,), jnp.int32),
            pltpu.VMEM((_B_PER_W,), jnp.int32),
            pltpu.VMEM((_NBUF, _EMB_DIM, 128), jnp.float32),
            pltpu.VMEM((_NBUF, _EMB_DIM, 128), jnp.float32),
            pltpu.VMEM((_B_PER_W,), jnp.float32),
            pltpu.VMEM((_B_PER_W,), jnp.float32),
            pltpu.VMEM((256,), jnp.float32),
            pltpu.VMEM((_B_PER_W,), jnp.float32),
            pltpu.SemaphoreType.DMA((_NBUF,)),
            pltpu.SemaphoreType.DMA((_NBUF,)),
            pltpu.SemaphoreType.DMA,
        ],
    )
    return run(uidx, iidx, ut, it, ub, ib)


# speedup vs baseline: 1.0104x; 1.0104x over previous
import jax
import jax.numpy as jnp
from jax import lax
from jax.experimental import pallas as pl
from jax.experimental.pallas import tpu as pltpu
from jax.experimental.pallas import tpu_sc as plsc

_EMB_DIM = 32
_BATCH = 16384
_NC = 2
_NS = 16
_NW = _NC * _NS
_B_PER_W = _BATCH // _NW
_CHUNK = 128
_NCHUNK = _B_PER_W // _CHUNK
_GROUPS = _B_PER_W // 16
_NBUF = 8


def _mf_body(uidx_hbm, iidx_hbm, ut_hbm, it_hbm, ubias_hbm, ibias_hbm,
             out_hbm,
             uidx_v, iidx_v, ublk, iblk, bu_v, bi_v, pbuf, out_v,
             usem, isem, bsem):
    wid = lax.axis_index("s") * _NC + lax.axis_index("c")

    pltpu.sync_copy(uidx_hbm.at[wid], uidx_v)
    pltpu.sync_copy(iidx_hbm.at[wid], iidx_v)

    bias_copies = []
    for j in range(_NCHUNK):
        sl = pl.ds(j * _CHUNK, _CHUNK)
        bias_copies.append(
            pltpu.async_copy(ubias_hbm.at[uidx_v.at[sl]], bu_v.at[sl], bsem))
        bias_copies.append(
            pltpu.async_copy(ibias_hbm.at[iidx_v.at[sl]], bi_v.at[sl], bsem))

    iota16 = lax.iota(jnp.int32, 16)

    def _issue(ucol, icol, slot):
        pltpu.async_copy(ut_hbm.at[:, pl.ds(pl.multiple_of(ucol, 128), 128)],
                         ublk.at[slot], usem.at[slot])
        pltpu.async_copy(it_hbm.at[:, pl.ds(pl.multiple_of(icol, 128), 128)],
                         iblk.at[slot], isem.at[slot])

    def _drain(slot):
        pltpu.make_async_copy(ut_hbm.at[:, pl.ds(0, 128)], ublk.at[slot],
                              usem.at[slot]).wait()
        pltpu.make_async_copy(it_hbm.at[:, pl.ds(0, 128)], iblk.at[slot],
                              isem.at[slot]).wait()

    def _cols_lanes(g):
        uvec = uidx_v[pl.ds(g * 16, 16)]
        ivec = iidx_v[pl.ds(g * 16, 16)]
        return ((uvec >> 7) * 128, uvec & 127, (ivec >> 7) * 128, ivec & 127)

    ucol0, _, icol0, _ = _cols_lanes(0)
    for b in range(_NBUF):
        _issue(ucol0[b], icol0[b], b % _NBUF)

    def group_body(g, carry):
        ucolv, ulanev, icolv, ilanev = _cols_lanes(g)
        for b in range(16):
            slot = b % _NBUF
            _drain(slot)
            ul = jnp.broadcast_to(ulanev[b], (16,))
            il = jnp.broadcast_to(ilanev[b], (16,))
            u0 = plsc.load_gather(ublk.at[slot], [iota16, ul])
            u1 = plsc.load_gather(ublk.at[slot], [iota16 + 16, ul])
            i0 = plsc.load_gather(iblk.at[slot], [iota16, il])
            i1 = plsc.load_gather(iblk.at[slot], [iota16 + 16, il])
            pbuf[pl.ds(b * 16, 16)] = u0 * i0 + u1 * i1

            if b < 16 - _NBUF:
                _issue(ucolv[b + _NBUF], icolv[b + _NBUF], slot)
            else:

                @pl.when(g < _GROUPS - 1)
                def _():
                    ucolvn, _, icolvn, _ = _cols_lanes(g + 1)
                    _issue(ucolvn[b + _NBUF - 16], icolvn[b + _NBUF - 16],
                           slot)

        acc = bu_v[pl.ds(g * 16, 16)] + bi_v[pl.ds(g * 16, 16)]
        for c in range(16):
            acc = acc + plsc.load_gather(pbuf, [iota16 * 16 + c])
        out_v[pl.ds(g * 16, 16)] = acc
        return carry

    for c in bias_copies:
        c.wait()

    lax.fori_loop(0, _GROUPS, group_body, 0)

    pltpu.sync_copy(out_v, out_hbm.at[pl.ds(wid * _B_PER_W, _B_PER_W)])


def kernel(edge_index, user_emb, item_emb, user_bias, item_bias):
    ut = user_emb.T
    it = item_emb.T
    uidx = edge_index[0].astype(jnp.int32).reshape(_NW, _B_PER_W)
    iidx = edge_index[1].astype(jnp.int32).reshape(_NW, _B_PER_W)
    ub = user_bias.reshape(-1)
    ib = item_bias.reshape(-1)

    mesh = plsc.VectorSubcoreMesh(core_axis_name="c", subcore_axis_name="s")
    run = pl.kernel(
        _mf_body,
        mesh=mesh,
        compiler_params=pltpu.CompilerParams(
            needs_layout_passes=False,
            use_tc_tiling_on_sc=True,
        ),
        out_type=jax.ShapeDtypeStruct((_BATCH,), jnp.float32),
        scratch_types=[
            pltpu.VMEM((_B_PER_W,), jnp.int32),
            pltpu.VMEM((_B_PER_W,), jnp.int32),
            pltpu.VMEM((_NBUF, _EMB_DIM, 128), jnp.float32),
            pltpu.VMEM((_NBUF, _EMB_DIM, 128), jnp.float32),
            pltpu.VMEM((_B_PER_W,), jnp.float32),
            pltpu.VMEM((_B_PER_W,), jnp.float32),
            pltpu.VMEM((256,), jnp.float32),
            pltpu.VMEM((_B_PER_W,), jnp.float32),
            pltpu.SemaphoreType.DMA((_NBUF,)),
            pltpu.SemaphoreType.DMA((_NBUF,)),
            pltpu.SemaphoreType.DMA,
        ],
    )
    return run(uidx, iidx, ut, it, ub, ib)
